# baseline (speedup 1.0000x reference)
import functools

import jax
import jax.numpy as jnp
from jax import lax
from jax.experimental import pallas as pl
from jax.experimental.pallas import tpu as pltpu
from jax.experimental.pallas import tpu_sc as plsc

B, L = 4096, 200
ITEM_DIM, GENRE_DIM = 64, 32
OUT_DIM = 128
TOK = B * L

NC, NS = 2, 16
NW = NC * NS
PER_W = TOK // NW
CHUNK = 128
N_CHUNKS = PER_W // CHUNK


def _sc_gather(item_hbm, genre_hbm, iid_hbm, gid_hbm,
               out_i_hbm, out_g_hbm,
               iidx_v, gidx_v, irows_v, grows_v, sem_i, sem_g):
    wid = lax.axis_index("s") * NC + lax.axis_index("c")
    base = wid * PER_W

    def body(j, carry):
        off = base + j * CHUNK
        pltpu.sync_copy(iid_hbm.at[pl.ds(off, CHUNK)], iidx_v)
        pltpu.sync_copy(gid_hbm.at[pl.ds(off, CHUNK)], gidx_v)
        cp_i = pltpu.async_copy(item_hbm.at[iidx_v], irows_v, sem_i)
        cp_g = pltpu.async_copy(genre_hbm.at[gidx_v], grows_v, sem_g)
        cp_i.wait()
        cp_g.wait()
        pltpu.sync_copy(irows_v, out_i_hbm.at[pl.ds(off, CHUNK)])
        pltpu.sync_copy(grows_v, out_g_hbm.at[pl.ds(off, CHUNK)])
        return carry

    lax.fori_loop(0, N_CHUNKS, body, 0)


def _tc_body(iid_ref, emb_i_ref, emb_g_ref, wi_ref, wg_ref, b_ref, pos_ref,
             out_ref):
    row = iid_ref[0]
    mask = (row != 0).astype(jnp.float32)
    ii = lax.broadcasted_iota(jnp.int32, (L, L), 0)
    jj = lax.broadcasted_iota(jnp.int32, (L, L), 1)
    dmask = jnp.where(ii == jj, mask, 0.0)
    ei = jnp.dot(dmask, emb_i_ref[...], preferred_element_type=jnp.float32)
    acc = jnp.dot(ei, wi_ref[...], preferred_element_type=jnp.float32)
    acc = acc + jnp.dot(emb_g_ref[...], wg_ref[...],
                        preferred_element_type=jnp.float32)
    out_ref[...] = acc + b_ref[...] + pos_ref[...]


def kernel(hist_item_id, hist_genre_id, item_table, genre_table, W, b,
           pos_table):
    iid_flat = hist_item_id.reshape(TOK)
    gid_flat = hist_genre_id.reshape(TOK)
    gt = genre_table.at[0].set(0.0)

    mesh = plsc.VectorSubcoreMesh(core_axis_name="c", subcore_axis_name="s")
    sc_gather = pl.kernel(
        _sc_gather,
        out_type=[
            jax.ShapeDtypeStruct((TOK, ITEM_DIM), jnp.float32),
            jax.ShapeDtypeStruct((TOK, GENRE_DIM), jnp.float32),
        ],
        mesh=mesh,
        scratch_types=[
            pltpu.VMEM((CHUNK,), jnp.int32),
            pltpu.VMEM((CHUNK,), jnp.int32),
            pltpu.VMEM((CHUNK, ITEM_DIM), jnp.float32),
            pltpu.VMEM((CHUNK, GENRE_DIM), jnp.float32),
            pltpu.SemaphoreType.DMA,
            pltpu.SemaphoreType.DMA,
        ],
        compiler_params=pltpu.CompilerParams(use_tc_tiling_on_sc=False),
    )
    emb_i, emb_g = sc_gather(item_table, gt, iid_flat, gid_flat)

    iid3 = iid_flat.reshape(B, 1, L)
    w_i = W[:ITEM_DIM]
    w_g = W[ITEM_DIM:]
    b2 = b.reshape(1, OUT_DIM)

    out = pl.pallas_call(
        _tc_body,
        grid=(B,),
        in_specs=[
            pl.BlockSpec((1, 1, L), lambda i: (i, 0, 0)),
            pl.BlockSpec((L, ITEM_DIM), lambda i: (i, 0)),
            pl.BlockSpec((L, GENRE_DIM), lambda i: (i, 0)),
            pl.BlockSpec((ITEM_DIM, OUT_DIM), lambda i: (0, 0)),
            pl.BlockSpec((GENRE_DIM, OUT_DIM), lambda i: (0, 0)),
            pl.BlockSpec((1, OUT_DIM), lambda i: (0, 0)),
            pl.BlockSpec((L, OUT_DIM), lambda i: (0, 0)),
        ],
        out_specs=pl.BlockSpec((L, OUT_DIM), lambda i: (i, 0)),
        out_shape=jax.ShapeDtypeStruct((TOK, OUT_DIM), jnp.float32),
    )(iid3, emb_i, emb_g, w_i, w_g, b2, pos_table)

    return out.reshape(B, L, OUT_DIM)


# speedup vs baseline: 1.8611x; 1.8611x over previous
import jax
import jax.numpy as jnp
from jax import lax
from jax.experimental import pallas as pl
from jax.experimental.pallas import tpu as pltpu
from jax.experimental.pallas import tpu_sc as plsc

B, L = 4096, 200
ITEM_DIM, GENRE_DIM = 64, 32
OUT_DIM = 128
TOK = B * L

NC, NS = 2, 16
NW = NC * NS
PER_W = TOK // NW
CHUNK = 128
N_CHUNKS = PER_W // CHUNK
SENT = -1

BR = 8
T_BLK = BR * L


def _sc_gather(item_hbm, genre_hbm, iid_hbm, gid_hbm, zrow_hbm,
               out_i_hbm, out_g_hbm,
               iidx_v, gidx_v, fidx_v, irows_v, grows_v, sem_i, sem_g, sem_f):
    wid = lax.axis_index("s") * NC + lax.axis_index("c")
    base = wid * PER_W

    def body(j, carry):
        off = base + j * CHUNK
        pltpu.sync_copy(iid_hbm.at[pl.ds(off, CHUNK)], iidx_v)
        pltpu.sync_copy(gid_hbm.at[pl.ds(off, CHUNK)], gidx_v)
        cp_i = pltpu.async_copy(item_hbm.at[iidx_v], irows_v, sem_i)
        cp_g = pltpu.async_copy(genre_hbm.at[gidx_v], grows_v, sem_g)
        for k in range(CHUNK // 16):
            v = iidx_v[pl.ds(k * 16, 16)]
            fidx_v[pl.ds(k * 16, 16)] = jnp.where(
                v == 0, jnp.zeros_like(v), jnp.full_like(v, SENT))
        cp_i.wait()
        cp_f = pltpu.async_copy(
            zrow_hbm.at[plsc.Indices(fidx_v, ignored_value=SENT)],
            irows_v, sem_f)
        cp_f.wait()
        cp_g.wait()
        pltpu.sync_copy(irows_v, out_i_hbm.at[pl.ds(off, CHUNK)])
        pltpu.sync_copy(grows_v, out_g_hbm.at[pl.ds(off, CHUNK)])
        return carry

    lax.fori_loop(0, N_CHUNKS, body, 0)


def _tc_body(emb_i_ref, emb_g_ref, wi_ref, wg_ref, b_ref, pos_ref, out_ref):
    acc = jnp.dot(emb_i_ref[...], wi_ref[...],
                  preferred_element_type=jnp.float32)
    acc = acc + jnp.dot(emb_g_ref[...], wg_ref[...],
                        preferred_element_type=jnp.float32)
    out_ref[...] = acc + b_ref[...] + pos_ref[...]


def kernel(hist_item_id, hist_genre_id, item_table, genre_table, W, b,
           pos_table):
    iid_flat = hist_item_id.reshape(TOK)
    gid_flat = hist_genre_id.reshape(TOK)
    gt = genre_table.at[0].set(0.0)
    zrow = jnp.zeros((8, ITEM_DIM), dtype=jnp.float32)

    mesh = plsc.VectorSubcoreMesh(core_axis_name="c", subcore_axis_name="s")
    sc_gather = pl.kernel(
        _sc_gather,
        out_type=[
            jax.ShapeDtypeStruct((TOK, ITEM_DIM), jnp.float32),
            jax.ShapeDtypeStruct((TOK, GENRE_DIM), jnp.float32),
        ],
        mesh=mesh,
        scratch_types=[
            pltpu.VMEM((CHUNK,), jnp.int32),
            pltpu.VMEM((CHUNK,), jnp.int32),
            pltpu.VMEM((CHUNK,), jnp.int32),
            pltpu.VMEM((CHUNK, ITEM_DIM), jnp.float32),
            pltpu.VMEM((CHUNK, GENRE_DIM), jnp.float32),
            pltpu.SemaphoreType.DMA,
            pltpu.SemaphoreType.DMA,
            pltpu.SemaphoreType.DMA,
        ],
        compiler_params=pltpu.CompilerParams(use_tc_tiling_on_sc=False),
    )
    emb_i, emb_g = sc_gather(item_table, gt, iid_flat, gid_flat, zrow)

    w_i = W[:ITEM_DIM]
    w_g = W[ITEM_DIM:]
    b2 = b.reshape(1, OUT_DIM)
    pos_blk = jnp.tile(pos_table, (BR, 1))

    out = pl.pallas_call(
        _tc_body,
        grid=(TOK // T_BLK,),
        in_specs=[
            pl.BlockSpec((T_BLK, ITEM_DIM), lambda i: (i, 0)),
            pl.BlockSpec((T_BLK, GENRE_DIM), lambda i: (i, 0)),
            pl.BlockSpec((ITEM_DIM, OUT_DIM), lambda i: (0, 0)),
            pl.BlockSpec((GENRE_DIM, OUT_DIM), lambda i: (0, 0)),
            pl.BlockSpec((1, OUT_DIM), lambda i: (0, 0)),
            pl.BlockSpec((T_BLK, OUT_DIM), lambda i: (0, 0)),
        ],
        out_specs=pl.BlockSpec((T_BLK, OUT_DIM), lambda i: (i, 0)),
        out_shape=jax.ShapeDtypeStruct((TOK, OUT_DIM), jnp.float32),
    )(emb_i, emb_g, w_i, w_g, b2, pos_blk)

    return out.reshape(B, L, OUT_DIM)


# speedup vs baseline: 2.6210x; 1.4083x over previous
import jax
import jax.numpy as jnp
from jax import lax
from jax.experimental import pallas as pl
from jax.experimental.pallas import tpu as pltpu
from jax.experimental.pallas import tpu_sc as plsc

B, L = 4096, 200
ITEM_DIM, GENRE_DIM = 64, 32
OUT_DIM = 128
TOK = B * L

NC, NS = 2, 16
NW = NC * NS
PER_W = TOK // NW
CHUNK = 128
N_CHUNKS = PER_W // CHUNK
SENT = -1

BR = 8
T_BLK = BR * L


def _sc_gather(item_hbm, genre_hbm, iid_hbm, gid_hbm, zrow_hbm,
               out_hbm,
               iidx_v, gidx_v, fidx_v, irows_v, grows_v, sem_i, sem_g, sem_f):
    wid = lax.axis_index("s") * NC + lax.axis_index("c")
    base = wid * PER_W

    def body(j, carry):
        off = base + j * CHUNK
        pltpu.sync_copy(iid_hbm.at[pl.ds(off, CHUNK)], iidx_v)
        pltpu.sync_copy(gid_hbm.at[pl.ds(off, CHUNK)], gidx_v)
        cp_i = pltpu.async_copy(item_hbm.at[iidx_v], irows_v, sem_i)
        cp_g = pltpu.async_copy(genre_hbm.at[gidx_v], grows_v, sem_g)
        for k in range(CHUNK // 16):
            v = iidx_v[pl.ds(k * 16, 16)]
            fidx_v[pl.ds(k * 16, 16)] = jnp.where(
                v == 0, jnp.zeros_like(v), jnp.full_like(v, SENT))
        cp_i.wait()
        cp_f = pltpu.async_copy(
            zrow_hbm.at[plsc.Indices(fidx_v, ignored_value=SENT)],
            irows_v, sem_f)
        cp_f.wait()
        cp_g.wait()
        pltpu.sync_copy(irows_v,
                        out_hbm.at[pl.ds(off, CHUNK), pl.ds(0, ITEM_DIM)])
        pltpu.sync_copy(grows_v,
                        out_hbm.at[pl.ds(off, CHUNK),
                                   pl.ds(ITEM_DIM, GENRE_DIM)])
        return carry

    lax.fori_loop(0, N_CHUNKS, body, 0)


def _tc_body(emb_ref, w_ref, b_ref, pos_ref, out_ref):
    e = emb_ref[:, :ITEM_DIM + GENRE_DIM]
    acc = jnp.dot(e, w_ref[...], preferred_element_type=jnp.float32)
    out_ref[...] = acc + b_ref[...] + pos_ref[...]


def kernel(hist_item_id, hist_genre_id, item_table, genre_table, W, b,
           pos_table):
    iid_flat = hist_item_id.reshape(TOK)
    gid_flat = hist_genre_id.reshape(TOK)
    gt = genre_table.at[0].set(0.0)
    zrow = jnp.zeros((8, ITEM_DIM), dtype=jnp.float32)

    mesh = plsc.VectorSubcoreMesh(core_axis_name="c", subcore_axis_name="s")
    sc_gather = pl.kernel(
        _sc_gather,
        out_type=[
            jax.ShapeDtypeStruct((TOK, 128), jnp.float32),
        ],
        mesh=mesh,
        scratch_types=[
            pltpu.VMEM((CHUNK,), jnp.int32),
            pltpu.VMEM((CHUNK,), jnp.int32),
            pltpu.VMEM((CHUNK,), jnp.int32),
            pltpu.VMEM((CHUNK, ITEM_DIM), jnp.float32),
            pltpu.VMEM((CHUNK, GENRE_DIM), jnp.float32),
            pltpu.SemaphoreType.DMA,
            pltpu.SemaphoreType.DMA,
            pltpu.SemaphoreType.DMA,
        ],
        compiler_params=pltpu.CompilerParams(use_tc_tiling_on_sc=False),
    )
    (emb,) = sc_gather(item_table, gt, iid_flat, gid_flat, zrow)

    b2 = b.reshape(1, OUT_DIM)
    pos_blk = jnp.tile(pos_table, (BR, 1))

    out = pl.pallas_call(
        _tc_body,
        grid=(TOK // T_BLK,),
        in_specs=[
            pl.BlockSpec((T_BLK, 128), lambda i: (i, 0)),
            pl.BlockSpec((ITEM_DIM + GENRE_DIM, OUT_DIM), lambda i: (0, 0)),
            pl.BlockSpec((1, OUT_DIM), lambda i: (0, 0)),
            pl.BlockSpec((T_BLK, OUT_DIM), lambda i: (0, 0)),
        ],
        out_specs=pl.BlockSpec((T_BLK, OUT_DIM), lambda i: (i, 0)),
        out_shape=jax.ShapeDtypeStruct((TOK, OUT_DIM), jnp.float32),
    )(emb, W, b2, pos_blk)

    return out.reshape(B, L, OUT_DIM)


# speedup vs baseline: 3.1599x; 1.2056x over previous
import jax
import jax.numpy as jnp
from jax import lax
from jax.experimental import pallas as pl
from jax.experimental.pallas import tpu as pltpu
from jax.experimental.pallas import tpu_sc as plsc

B, L = 4096, 200
ITEM_DIM, GENRE_DIM = 64, 32
OUT_DIM = 128
TOK = B * L

NC, NS = 2, 16
NW = NC * NS
PER_W = TOK // NW
CHUNK = 128
N_CHUNKS = PER_W // CHUNK
NBUF = 4
SENT = -1

BR = 8
T_BLK = BR * L


def _sc_gather(item_hbm, genre_hbm, iid_hbm, gid_hbm, zrow_hbm,
               out_hbm, iidx_v, gidx_v, *scr):
    wid = lax.axis_index("s") * NC + lax.axis_index("c")
    base = wid * PER_W
    fidx = scr[0:NBUF]
    irows = scr[NBUF:2 * NBUF]
    grows = scr[2 * NBUF:3 * NBUF]
    sem_i = scr[3 * NBUF:4 * NBUF]
    sem_g = scr[4 * NBUF:5 * NBUF]
    sem_f = scr[5 * NBUF]

    pltpu.sync_copy(iid_hbm.at[pl.ds(base, PER_W)], iidx_v)
    pltpu.sync_copy(gid_hbm.at[pl.ds(base, PER_W)], gidx_v)

    def body(t, carry):
        j0 = t * NBUF
        cps = []
        for b in range(NBUF):
            loc = (j0 + b) * CHUNK
            cp_i = pltpu.async_copy(
                item_hbm.at[iidx_v.at[pl.ds(loc, CHUNK)]], irows[b], sem_i[b])
            cp_g = pltpu.async_copy(
                genre_hbm.at[gidx_v.at[pl.ds(loc, CHUNK)]], grows[b], sem_g[b])
            cps.append((cp_i, cp_g))
        outs = []
        for b in range(NBUF):
            loc = (j0 + b) * CHUNK
            off = base + loc
            cp_i, cp_g = cps[b]
            for k in range(CHUNK // 16):
                v = iidx_v[pl.ds(loc + k * 16, 16)]
                fidx[b][pl.ds(k * 16, 16)] = jnp.where(
                    v == 0, jnp.zeros_like(v), jnp.full_like(v, SENT))
            cp_i.wait()
            pltpu.async_copy(
                zrow_hbm.at[plsc.Indices(fidx[b], ignored_value=SENT)],
                irows[b], sem_f).wait()
            cp_g.wait()
            pltpu.sync_copy(
                irows[b], out_hbm.at[pl.ds(off, CHUNK), pl.ds(0, ITEM_DIM)])
            pltpu.sync_copy(
                grows[b],
                out_hbm.at[pl.ds(off, CHUNK), pl.ds(ITEM_DIM, GENRE_DIM)])
        del outs
        return carry

    lax.fori_loop(0, N_CHUNKS // NBUF, body, 0)


def _tc_body(emb_ref, w_ref, b_ref, pos_ref, out_ref):
    e = emb_ref[:, :ITEM_DIM + GENRE_DIM]
    acc = jnp.dot(e, w_ref[...], preferred_element_type=jnp.float32)
    out_ref[...] = acc + b_ref[...] + pos_ref[...]


def kernel(hist_item_id, hist_genre_id, item_table, genre_table, W, b,
           pos_table):
    iid_flat = hist_item_id.reshape(TOK)
    gid_flat = hist_genre_id.reshape(TOK)
    gt = genre_table.at[0].set(0.0)
    zrow = jnp.zeros((8, ITEM_DIM), dtype=jnp.float32)

    mesh = plsc.VectorSubcoreMesh(core_axis_name="c", subcore_axis_name="s")
    sc_gather = pl.kernel(
        _sc_gather,
        out_type=[
            jax.ShapeDtypeStruct((TOK, 128), jnp.float32),
        ],
        mesh=mesh,
        scratch_types=[
            pltpu.VMEM((PER_W,), jnp.int32),
            pltpu.VMEM((PER_W,), jnp.int32),
        ] + [pltpu.VMEM((CHUNK,), jnp.int32)] * NBUF
          + [pltpu.VMEM((CHUNK, ITEM_DIM), jnp.float32)] * NBUF
          + [pltpu.VMEM((CHUNK, GENRE_DIM), jnp.float32)] * NBUF
          + [pltpu.SemaphoreType.DMA] * NBUF
          + [pltpu.SemaphoreType.DMA] * NBUF
          + [pltpu.SemaphoreType.DMA],
        compiler_params=pltpu.CompilerParams(use_tc_tiling_on_sc=False),
    )
    (emb,) = sc_gather(item_table, gt, iid_flat, gid_flat, zrow)

    b2 = b.reshape(1, OUT_DIM)
    pos_blk = jnp.tile(pos_table, (BR, 1))

    out = pl.pallas_call(
        _tc_body,
        grid=(TOK // T_BLK,),
        in_specs=[
            pl.BlockSpec((T_BLK, 128), lambda i: (i, 0)),
            pl.BlockSpec((ITEM_DIM + GENRE_DIM, OUT_DIM), lambda i: (0, 0)),
            pl.BlockSpec((1, OUT_DIM), lambda i: (0, 0)),
            pl.BlockSpec((T_BLK, OUT_DIM), lambda i: (0, 0)),
        ],
        out_specs=pl.BlockSpec((T_BLK, OUT_DIM), lambda i: (i, 0)),
        out_shape=jax.ShapeDtypeStruct((TOK, OUT_DIM), jnp.float32),
    )(emb, W, b2, pos_blk)

    return out.reshape(B, L, OUT_DIM)


# speedup vs baseline: 3.2666x; 1.0338x over previous
import jax
import jax.numpy as jnp
from jax import lax
from jax.experimental import pallas as pl
from jax.experimental.pallas import tpu as pltpu
from jax.experimental.pallas import tpu_sc as plsc

B, L = 4096, 200
ITEM_DIM, GENRE_DIM = 64, 32
EMB_DIM = ITEM_DIM + GENRE_DIM
OUT_DIM = 128
TOK = B * L

NH = 2
TOK_H = TOK // NH

NC, NS = 2, 16
NW = NC * NS
PER_W = TOK_H // NW
CHUNK = 128
N_CHUNKS = PER_W // CHUNK
NBUF = 4
SENT = -1

BR = 8
T_BLK = BR * L
GRID_H = TOK_H // T_BLK


def _sc_gather(item_hbm, genre_hbm, iid_hbm, gid_hbm, zrow_hbm,
               out_hbm, iidx_v, gidx_v, *scr):
    wid = lax.axis_index("s") * NC + lax.axis_index("c")
    base = wid * PER_W
    fidx = scr[0:NBUF]
    irows = scr[NBUF:2 * NBUF]
    grows = scr[2 * NBUF:3 * NBUF]
    sem_i = scr[3 * NBUF:4 * NBUF]
    sem_g = scr[4 * NBUF:5 * NBUF]
    sem_f = scr[5 * NBUF]

    pltpu.sync_copy(iid_hbm.at[pl.ds(base, PER_W)], iidx_v)
    pltpu.sync_copy(gid_hbm.at[pl.ds(base, PER_W)], gidx_v)

    def body(t, carry):
        j0 = t * NBUF
        cps = []
        for b in range(NBUF):
            loc = (j0 + b) * CHUNK
            cp_i = pltpu.async_copy(
                item_hbm.at[iidx_v.at[pl.ds(loc, CHUNK)]], irows[b], sem_i[b])
            cp_g = pltpu.async_copy(
                genre_hbm.at[gidx_v.at[pl.ds(loc, CHUNK)]], grows[b], sem_g[b])
            cps.append((cp_i, cp_g))
        for b in range(NBUF):
            loc = (j0 + b) * CHUNK
            off = base + loc
            cp_i, cp_g = cps[b]
            for k in range(CHUNK // 16):
                v = iidx_v[pl.ds(loc + k * 16, 16)]
                fidx[b][pl.ds(k * 16, 16)] = jnp.where(
                    v == 0, jnp.zeros_like(v), jnp.full_like(v, SENT))
            cp_i.wait()
            pltpu.async_copy(
                zrow_hbm.at[plsc.Indices(fidx[b], ignored_value=SENT)],
                irows[b], sem_f).wait()
            cp_g.wait()
            pltpu.sync_copy(
                irows[b], out_hbm.at[pl.ds(off, CHUNK), pl.ds(0, ITEM_DIM)])
            pltpu.sync_copy(
                grows[b],
                out_hbm.at[pl.ds(off, CHUNK), pl.ds(ITEM_DIM, GENRE_DIM)])
        return carry

    lax.fori_loop(0, N_CHUNKS // NBUF, body, 0)


def _tc_body(emb_ref, w_ref, b_ref, pos_ref, out_ref):
    e = emb_ref[:, :EMB_DIM]
    acc = jnp.dot(e, w_ref[...], preferred_element_type=jnp.float32)
    out_ref[...] = acc + b_ref[...] + pos_ref[...]


def _tc_body_chained(emb_ref, w_ref, b_ref, pos_ref, prev_ref, out_ref):
    del prev_ref
    e = emb_ref[:, :EMB_DIM]
    acc = jnp.dot(e, w_ref[...], preferred_element_type=jnp.float32)
    out_ref[...] = acc + b_ref[...] + pos_ref[...]


def kernel(hist_item_id, hist_genre_id, item_table, genre_table, W, b,
           pos_table):
    iid_flat = hist_item_id.reshape(TOK)
    gid_flat = hist_genre_id.reshape(TOK)
    gt = genre_table.at[0].set(0.0)
    zrow = jnp.zeros((8, ITEM_DIM), dtype=jnp.float32)

    mesh = plsc.VectorSubcoreMesh(core_axis_name="c", subcore_axis_name="s")
    sc_gather = pl.kernel(
        _sc_gather,
        out_type=[
            jax.ShapeDtypeStruct((TOK_H, 128), jnp.float32),
        ],
        mesh=mesh,
        scratch_types=[
            pltpu.VMEM((PER_W,), jnp.int32),
            pltpu.VMEM((PER_W,), jnp.int32),
        ] + [pltpu.VMEM((CHUNK,), jnp.int32)] * NBUF
          + [pltpu.VMEM((CHUNK, ITEM_DIM), jnp.float32)] * NBUF
          + [pltpu.VMEM((CHUNK, GENRE_DIM), jnp.float32)] * NBUF
          + [pltpu.SemaphoreType.DMA] * NBUF
          + [pltpu.SemaphoreType.DMA] * NBUF
          + [pltpu.SemaphoreType.DMA],
        compiler_params=pltpu.CompilerParams(use_tc_tiling_on_sc=False),
    )
    embs = []
    for h in range(NH):
        (emb_h,) = sc_gather(
            item_table, gt,
            lax.slice_in_dim(iid_flat, h * TOK_H, (h + 1) * TOK_H),
            lax.slice_in_dim(gid_flat, h * TOK_H, (h + 1) * TOK_H),
            zrow)
        embs.append(emb_h)

    b2 = b.reshape(1, OUT_DIM)
    pos_blk = jnp.tile(pos_table, (BR, 1))

    out = None
    for h in range(NH):
        emb_specs = [
            pl.BlockSpec((T_BLK, 128), lambda i: (i, 0)),
            pl.BlockSpec((EMB_DIM, OUT_DIM), lambda i: (0, 0)),
            pl.BlockSpec((1, OUT_DIM), lambda i: (0, 0)),
            pl.BlockSpec((T_BLK, OUT_DIM), lambda i: (0, 0)),
        ]
        out_spec = pl.BlockSpec((T_BLK, OUT_DIM),
                                lambda i, h=h: (i + h * GRID_H, 0))
        if h == 0:
            out = pl.pallas_call(
                _tc_body,
                grid=(GRID_H,),
                in_specs=emb_specs,
                out_specs=out_spec,
                out_shape=jax.ShapeDtypeStruct((TOK, OUT_DIM), jnp.float32),
            )(embs[h], W, b2, pos_blk)
        else:
            out = pl.pallas_call(
                _tc_body_chained,
                grid=(GRID_H,),
                in_specs=emb_specs + [pl.BlockSpec(memory_space=pl.ANY)],
                out_specs=out_spec,
                out_shape=jax.ShapeDtypeStruct((TOK, OUT_DIM), jnp.float32),
                input_output_aliases={4: 0},
            )(embs[h], W, b2, pos_blk, out)

    return out.reshape(B, L, OUT_DIM)


# speedup vs baseline: 3.5977x; 1.1013x over previous
import jax
import jax.numpy as jnp
from jax import lax
from jax.experimental import pallas as pl
from jax.experimental.pallas import tpu as pltpu
from jax.experimental.pallas import tpu_sc as plsc

B, L = 4096, 200
ITEM_DIM, GENRE_DIM = 64, 32
EMB_DIM = ITEM_DIM + GENRE_DIM
OUT_DIM = 128
TOK = B * L

NH = 2
TOK_H = TOK // NH

NC, NS = 2, 16
NW = NC * NS
PER_W = TOK_H // NW
CHUNK = 128
N_CHUNKS = PER_W // CHUNK
NBUF = 4
SENT = -1

BR = 16
T_BLK = BR * L
GRID_H = TOK_H // T_BLK


def _sc_gather(item_hbm, genre_hbm, iid_hbm, gid_hbm, zrow_hbm,
               out_hbm, iidx_v, gidx_v, *scr):
    wid = lax.axis_index("s") * NC + lax.axis_index("c")
    base = wid * PER_W
    fidx = scr[0:NBUF]
    irows = scr[NBUF:2 * NBUF]
    grows = scr[2 * NBUF:3 * NBUF]
    sem_i = scr[3 * NBUF:4 * NBUF]
    sem_g = scr[4 * NBUF:5 * NBUF]
    sem_f = scr[5 * NBUF]

    pltpu.sync_copy(iid_hbm.at[pl.ds(base, PER_W)], iidx_v)
    pltpu.sync_copy(gid_hbm.at[pl.ds(base, PER_W)], gidx_v)

    def body(t, carry):
        j0 = t * NBUF
        cps = []
        for b in range(NBUF):
            loc = (j0 + b) * CHUNK
            cp_i = pltpu.async_copy(
                item_hbm.at[iidx_v.at[pl.ds(loc, CHUNK)]], irows[b], sem_i[b])
            cp_g = pltpu.async_copy(
                genre_hbm.at[gidx_v.at[pl.ds(loc, CHUNK)]], grows[b], sem_g[b])
            cps.append((cp_i, cp_g))
        for b in range(NBUF):
            loc = (j0 + b) * CHUNK
            off = base + loc
            cp_i, cp_g = cps[b]
            for k in range(CHUNK // 16):
                v = iidx_v[pl.ds(loc + k * 16, 16)]
                fidx[b][pl.ds(k * 16, 16)] = jnp.where(
                    v == 0, jnp.zeros_like(v), jnp.full_like(v, SENT))
            cp_i.wait()
            pltpu.async_copy(
                zrow_hbm.at[plsc.Indices(fidx[b], ignored_value=SENT)],
                irows[b], sem_f).wait()
            cp_g.wait()
            pltpu.sync_copy(
                irows[b], out_hbm.at[pl.ds(off, CHUNK), pl.ds(0, ITEM_DIM)])
            pltpu.sync_copy(
                grows[b],
                out_hbm.at[pl.ds(off, CHUNK), pl.ds(ITEM_DIM, GENRE_DIM)])
        return carry

    lax.fori_loop(0, N_CHUNKS // NBUF, body, 0)


def _tc_body(emb_ref, w_ref, b_ref, pos_ref, out_ref):
    e = emb_ref[:, :EMB_DIM]
    acc = jnp.dot(e, w_ref[...], preferred_element_type=jnp.float32)
    out_ref[...] = acc + b_ref[...] + pos_ref[...]


def _tc_body_chained(emb_ref, w_ref, b_ref, pos_ref, prev_ref, out_ref):
    del prev_ref
    e = emb_ref[:, :EMB_DIM]
    acc = jnp.dot(e, w_ref[...], preferred_element_type=jnp.float32)
    out_ref[...] = acc + b_ref[...] + pos_ref[...]


def kernel(hist_item_id, hist_genre_id, item_table, genre_table, W, b,
           pos_table):
    iid_flat = hist_item_id.reshape(TOK)
    gid_flat = hist_genre_id.reshape(TOK)
    gt = genre_table.at[0].set(0.0)
    zrow = jnp.zeros((8, ITEM_DIM), dtype=jnp.float32)

    mesh = plsc.VectorSubcoreMesh(core_axis_name="c", subcore_axis_name="s")
    sc_gather = pl.kernel(
        _sc_gather,
        out_type=[
            jax.ShapeDtypeStruct((TOK_H, 128), jnp.float32),
        ],
        mesh=mesh,
        scratch_types=[
            pltpu.VMEM((PER_W,), jnp.int32),
            pltpu.VMEM((PER_W,), jnp.int32),
        ] + [pltpu.VMEM((CHUNK,), jnp.int32)] * NBUF
          + [pltpu.VMEM((CHUNK, ITEM_DIM), jnp.float32)] * NBUF
          + [pltpu.VMEM((CHUNK, GENRE_DIM), jnp.float32)] * NBUF
          + [pltpu.SemaphoreType.DMA] * NBUF
          + [pltpu.SemaphoreType.DMA] * NBUF
          + [pltpu.SemaphoreType.DMA],
        compiler_params=pltpu.CompilerParams(use_tc_tiling_on_sc=False),
    )
    embs = []
    for h in range(NH):
        (emb_h,) = sc_gather(
            item_table, gt,
            lax.slice_in_dim(iid_flat, h * TOK_H, (h + 1) * TOK_H),
            lax.slice_in_dim(gid_flat, h * TOK_H, (h + 1) * TOK_H),
            zrow)
        embs.append(emb_h)

    b2 = b.reshape(1, OUT_DIM)
    pos_blk = jnp.tile(pos_table, (BR, 1))

    out = None
    for h in range(NH):
        emb_specs = [
            pl.BlockSpec((T_BLK, 128), lambda i: (i, 0)),
            pl.BlockSpec((EMB_DIM, OUT_DIM), lambda i: (0, 0)),
            pl.BlockSpec((1, OUT_DIM), lambda i: (0, 0)),
            pl.BlockSpec((T_BLK, OUT_DIM), lambda i: (0, 0)),
        ]
        out_spec = pl.BlockSpec((T_BLK, OUT_DIM),
                                lambda i, h=h: (i + h * GRID_H, 0))
        if h == 0:
            out = pl.pallas_call(
                _tc_body,
                grid=(GRID_H,),
                in_specs=emb_specs,
                out_specs=out_spec,
                out_shape=jax.ShapeDtypeStruct((TOK, OUT_DIM), jnp.float32),
            )(embs[h], W, b2, pos_blk)
        else:
            out = pl.pallas_call(
                _tc_body_chained,
                grid=(GRID_H,),
                in_specs=emb_specs + [pl.BlockSpec(memory_space=pl.ANY)],
                out_specs=out_spec,
                out_shape=jax.ShapeDtypeStruct((TOK, OUT_DIM), jnp.float32),
                input_output_aliases={4: 0},
            )(embs[h], W, b2, pos_blk, out)

    return out.reshape(B, L, OUT_DIM)


# speedup vs baseline: 3.7376x; 1.0389x over previous
import jax
import jax.numpy as jnp
from jax import lax
from jax.experimental import pallas as pl
from jax.experimental.pallas import tpu as pltpu
from jax.experimental.pallas import tpu_sc as plsc

B, L = 4096, 200
ITEM_DIM, GENRE_DIM = 64, 32
EMB_DIM = ITEM_DIM + GENRE_DIM
OUT_DIM = 128
TOK = B * L

NH = 2
TOK_H = TOK // NH

NC, NS = 2, 16
NW = NC * NS
PER_W = TOK_H // NW
CHUNK = 128
N_CHUNKS = PER_W // CHUNK
NBUF = 4
SENT = -1

BR = 32
T_BLK = BR * L
GRID_H = TOK_H // T_BLK


def _sc_gather(item_hbm, genre_hbm, iid_hbm, gid_hbm, zrow_hbm,
               out_hbm, iidx_v, gidx_v, *scr):
    wid = lax.axis_index("s") * NC + lax.axis_index("c")
    base = wid * PER_W
    fidx = scr[0:NBUF]
    irows = scr[NBUF:2 * NBUF]
    grows = scr[2 * NBUF:3 * NBUF]
    sem_i = scr[3 * NBUF:4 * NBUF]
    sem_g = scr[4 * NBUF:5 * NBUF]
    sem_f = scr[5 * NBUF]

    pltpu.sync_copy(iid_hbm.at[pl.ds(base, PER_W)], iidx_v)
    pltpu.sync_copy(gid_hbm.at[pl.ds(base, PER_W)], gidx_v)

    def body(t, carry):
        j0 = t * NBUF
        cps = []
        for b in range(NBUF):
            loc = (j0 + b) * CHUNK
            cp_i = pltpu.async_copy(
                item_hbm.at[iidx_v.at[pl.ds(loc, CHUNK)]], irows[b], sem_i[b])
            cp_g = pltpu.async_copy(
                genre_hbm.at[gidx_v.at[pl.ds(loc, CHUNK)]], grows[b], sem_g[b])
            cps.append((cp_i, cp_g))
        for b in range(NBUF):
            loc = (j0 + b) * CHUNK
            off = base + loc
            cp_i, cp_g = cps[b]
            for k in range(CHUNK // 16):
                v = iidx_v[pl.ds(loc + k * 16, 16)]
                fidx[b][pl.ds(k * 16, 16)] = jnp.where(
                    v == 0, jnp.zeros_like(v), jnp.full_like(v, SENT))
            cp_i.wait()
            pltpu.async_copy(
                zrow_hbm.at[plsc.Indices(fidx[b], ignored_value=SENT)],
                irows[b], sem_f).wait()
            cp_g.wait()
            pltpu.sync_copy(
                irows[b], out_hbm.at[pl.ds(off, CHUNK), pl.ds(0, ITEM_DIM)])
            pltpu.sync_copy(
                grows[b],
                out_hbm.at[pl.ds(off, CHUNK), pl.ds(ITEM_DIM, GENRE_DIM)])
        return carry

    lax.fori_loop(0, N_CHUNKS // NBUF, body, 0)


def _tc_body(emb_ref, w_ref, b_ref, pos_ref, out_ref):
    e = emb_ref[:, :EMB_DIM]
    acc = jnp.dot(e, w_ref[...], preferred_element_type=jnp.float32)
    out_ref[...] = acc + b_ref[...] + pos_ref[...]


def _tc_body_chained(emb_ref, w_ref, b_ref, pos_ref, prev_ref, out_ref):
    del prev_ref
    e = emb_ref[:, :EMB_DIM]
    acc = jnp.dot(e, w_ref[...], preferred_element_type=jnp.float32)
    out_ref[...] = acc + b_ref[...] + pos_ref[...]


def kernel(hist_item_id, hist_genre_id, item_table, genre_table, W, b,
           pos_table):
    iid_flat = hist_item_id.reshape(TOK)
    gid_flat = hist_genre_id.reshape(TOK)
    gt = genre_table.at[0].set(0.0)
    zrow = jnp.zeros((8, ITEM_DIM), dtype=jnp.float32)

    mesh = plsc.VectorSubcoreMesh(core_axis_name="c", subcore_axis_name="s")
    sc_gather = pl.kernel(
        _sc_gather,
        out_type=[
            jax.ShapeDtypeStruct((TOK_H, 128), jnp.float32),
        ],
        mesh=mesh,
        scratch_types=[
            pltpu.VMEM((PER_W,), jnp.int32),
            pltpu.VMEM((PER_W,), jnp.int32),
        ] + [pltpu.VMEM((CHUNK,), jnp.int32)] * NBUF
          + [pltpu.VMEM((CHUNK, ITEM_DIM), jnp.float32)] * NBUF
          + [pltpu.VMEM((CHUNK, GENRE_DIM), jnp.float32)] * NBUF
          + [pltpu.SemaphoreType.DMA] * NBUF
          + [pltpu.SemaphoreType.DMA] * NBUF
          + [pltpu.SemaphoreType.DMA],
        compiler_params=pltpu.CompilerParams(use_tc_tiling_on_sc=False),
    )
    embs = []
    for h in range(NH):
        (emb_h,) = sc_gather(
            item_table, gt,
            lax.slice_in_dim(iid_flat, h * TOK_H, (h + 1) * TOK_H),
            lax.slice_in_dim(gid_flat, h * TOK_H, (h + 1) * TOK_H),
            zrow)
        embs.append(emb_h)

    b2 = b.reshape(1, OUT_DIM)
    pos_blk = jnp.tile(pos_table, (BR, 1))

    out = None
    for h in range(NH):
        emb_specs = [
            pl.BlockSpec((T_BLK, 128), lambda i: (i, 0)),
            pl.BlockSpec((EMB_DIM, OUT_DIM), lambda i: (0, 0)),
            pl.BlockSpec((1, OUT_DIM), lambda i: (0, 0)),
            pl.BlockSpec((T_BLK, OUT_DIM), lambda i: (0, 0)),
        ]
        out_spec = pl.BlockSpec((T_BLK, OUT_DIM),
                                lambda i, h=h: (i + h * GRID_H, 0))
        if h == 0:
            out = pl.pallas_call(
                _tc_body,
                grid=(GRID_H,),
                in_specs=emb_specs,
                out_specs=out_spec,
                out_shape=jax.ShapeDtypeStruct((TOK, OUT_DIM), jnp.float32),
            )(embs[h], W, b2, pos_blk)
        else:
            out = pl.pallas_call(
                _tc_body_chained,
                grid=(GRID_H,),
                in_specs=emb_specs + [pl.BlockSpec(memory_space=pl.ANY)],
                out_specs=out_spec,
                out_shape=jax.ShapeDtypeStruct((TOK, OUT_DIM), jnp.float32),
                input_output_aliases={4: 0},
            )(embs[h], W, b2, pos_blk, out)

    return out.reshape(B, L, OUT_DIM)
